# baseline (speedup 1.0000x reference)
import functools

import jax
import jax.numpy as jnp
from jax import lax
from jax.experimental import pallas as pl
from jax.experimental.pallas import tpu as pltpu
from jax.experimental.pallas import tpu_sc as plsc

N = 100000
D = 16
E = 3200000
NC = 2
NS = 16
NW = NC * NS
L = 16

NP = 100352
CH = 128
EPT = 100352
PE = NW * EPT
BPT = EPT // CH

KP = 4
WP = KP * CH
NWP = EPT // WP

KD = 8
WD = KD * CH
NWD = EPT // WD

KD2 = 2
WD2 = KD2 * CH
NWD2 = EPT // WD2

RPC = NP // NS
ZR = 224
RPT = NP // NW
CR = 784

_mesh = plsc.VectorSubcoreMesh(core_axis_name="c", subcore_axis_name="s")
_params = pltpu.CompilerParams(use_tc_tiling_on_sc=False, needs_layout_passes=False)


def _ids():
    c = lax.axis_index("c")
    s = lax.axis_index("s")
    return c, s, c * NS + s


def _fill(ref, rows, value):
    v = jnp.full((L,), value, jnp.float32)
    for r in range(rows):
        ref[r, :] = v


def _zero_acc(acc, zb, s):
    _fill(zb, ZR, 0.0)
    r0 = s * RPC
    for i in range(RPC // ZR):
        pltpu.sync_copy(zb.at[pl.ds(0, ZR)], acc.at[pl.ds(r0 + i * ZR, ZR)])


def _dump_acc(acc, partials, c, s):
    r0 = s * RPC
    pltpu.sync_copy(acc.at[pl.ds(r0, RPC)], partials.at[c, pl.ds(r0, RPC)])


@functools.partial(
    pl.kernel,
    out_type=jax.ShapeDtypeStruct((NC, NP), jnp.float32),
    mesh=_mesh,
    compiler_params=_params,
    scratch_types=[
        pltpu.VMEM_SHARED((NP,), jnp.float32),
        pltpu.VMEM((WD,), jnp.int32),
        pltpu.VMEM((WD,), jnp.int32),
        pltpu.VMEM((WD,), jnp.float32),
        pltpu.VMEM((RPC // 2,), jnp.float32),
        pltpu.SemaphoreType.DMA,
        pltpu.SemaphoreType.DMA,
    ],
)
def _deg_kernel(col1, partials, acc, cola, colb, ones, zb, ssema, ssemb):
    c, s, wid = _ids()
    for r in range(RPC // 2 // L):
        zb[pl.ds(r * L, L)] = jnp.zeros((L,), jnp.float32)
    r0 = s * RPC
    pltpu.sync_copy(zb, acc.at[pl.ds(r0, RPC // 2)])
    pltpu.sync_copy(zb, acc.at[pl.ds(r0 + RPC // 2, RPC // 2)])
    for r in range(WD // L):
        ones[pl.ds(r * L, L)] = jnp.full((L,), 1.0, jnp.float32)
    plsc.subcore_barrier()

    def load_idx(w, colx):
        base = wid * EPT + w * WD
        pltpu.sync_copy(col1.at[pl.ds(base, WD)], colx)

    def stage(w, colx, ssemx, coln, ssemn):
        @pl.when(w > 0)
        def _():
            pltpu.make_async_copy(ones, acc.at[coln], ssemn).wait()

        @pl.when(w + 1 < NWD)
        def _():
            load_idx(w + 1, coln)

        pltpu.async_copy(ones, acc.at[colx], ssemx, add=True)

    load_idx(0, cola)

    def pair(w2, carry):
        w = 2 * w2
        stage(w, cola, ssema, colb, ssemb)
        stage(w + 1, colb, ssemb, cola, ssema)
        return carry

    lax.fori_loop(0, NWD // 2, pair, 0)
    pltpu.make_async_copy(ones, acc.at[colb], ssemb).wait()
    plsc.subcore_barrier()
    pltpu.sync_copy(acc.at[pl.ds(r0, RPC)], partials.at[c, pl.ds(r0, RPC)])


@functools.partial(
    pl.kernel,
    out_type=jax.ShapeDtypeStruct((NC, NP, D), jnp.float32),
    mesh=_mesh,
    compiler_params=_params,
    scratch_types=[
        pltpu.VMEM_SHARED((NP, D), jnp.float32),
        pltpu.VMEM((WP,), jnp.int32),
        pltpu.VMEM((WP,), jnp.int32),
        pltpu.VMEM((WP,), jnp.int32),
        pltpu.VMEM((WP,), jnp.int32),
        pltpu.VMEM((WP, D), jnp.float32),
        pltpu.VMEM((WP, D), jnp.float32),
        pltpu.SemaphoreType.DMA,
        pltpu.SemaphoreType.DMA,
        pltpu.SemaphoreType.DMA,
        pltpu.SemaphoreType.DMA,
    ],
)
def _prop_kernel(row1, col1, ytab, partials,
                 acc, rowa, cola, rowb, colb, msga, msgb,
                 sema, semb, ssema, ssemb):
    c, s, wid = _ids()
    _zero_acc(acc, msga, s)
    plsc.subcore_barrier()

    def fetch(w, rowx, msgx, semx):
        base = wid * EPT + w * WP
        pltpu.sync_copy(row1.at[pl.ds(base, WP)], rowx)
        pltpu.async_copy(ytab.at[rowx], msgx, semx)

    def load_cols(w, colx):
        base = wid * EPT + w * WP
        pltpu.sync_copy(col1.at[pl.ds(base, WP)], colx)

    def stage(w, rowx, colx, msgx, semx, ssemx, rown, coln, msgn, semn, ssemn):
        @pl.when(w > 0)
        def _():
            pltpu.make_async_copy(msgn, acc.at[coln], ssemn).wait()

        @pl.when(w + 1 < NWP)
        def _():
            fetch(w + 1, rown, msgn, semn)

        pltpu.make_async_copy(ytab.at[pl.ds(0, WP)], msgx, semx).wait()
        load_cols(w, colx)
        pltpu.async_copy(msgx, acc.at[colx], ssemx, add=True)

    fetch(0, rowa, msga, sema)

    def pair(w2, carry):
        w = 2 * w2
        stage(w, rowa, cola, msga, sema, ssema, rowb, colb, msgb, semb, ssemb)
        stage(w + 1, rowb, colb, msgb, semb, ssemb, rowa, cola, msga, sema, ssema)
        return carry

    lax.fori_loop(0, NWP // 2, pair, 0)
    pltpu.make_async_copy(msgb, acc.at[colb], ssemb).wait()
    plsc.subcore_barrier()
    _dump_acc(acc, partials, c, s)


def _ew_loop(body, in_hbm, out_hbm, bufs):
    _, _, wid = _ids()
    n_in = len(in_hbm)
    in_bufs, out_bufs = bufs[:n_in], bufs[n_in:]
    for chunk in range(RPT // CR):
        r0 = wid * RPT + chunk * CR
        for ref, buf in zip(in_hbm, in_bufs):
            pltpu.sync_copy(ref.at[pl.ds(r0, CR)], buf)

        def row(i, carry):
            outs = body(i, *in_bufs)
            for o, b in zip(outs, out_bufs):
                b[i, :] = o
            return carry

        lax.fori_loop(0, CR, row, 0)
        for ref, buf in zip(out_hbm, out_bufs):
            pltpu.sync_copy(buf, ref.at[pl.ds(r0, CR)])


def _rsqrt(x):
    i = plsc.bitcast(x, jnp.int32)
    i = jnp.int32(0x5F3759DF) - lax.shift_right_logical(i, 1)
    y = plsc.bitcast(i, jnp.float32)
    half = x * 0.5
    for _ in range(4):
        y = y * (1.5 - half * y * y)
    return y


@functools.partial(
    pl.kernel,
    out_type=(
        jax.ShapeDtypeStruct((NP, D), jnp.float32),
        jax.ShapeDtypeStruct((NP, D), jnp.float32),
    ),
    mesh=_mesh,
    compiler_params=_params,
    scratch_types=[
        pltpu.VMEM((CR,), jnp.float32),
        pltpu.VMEM((CR,), jnp.float32),
        pltpu.VMEM((CR,), jnp.float32),
        pltpu.VMEM((CR, D), jnp.float32),
        pltpu.VMEM((CR, D), jnp.float32),
        pltpu.VMEM((CR, D), jnp.float32),
    ],
)
def _comb1_kernel(partials, emb, dinv, y0, *bufs):
    _, _, wid = _ids()
    pa, pb, dv1, pe, odinv, oy = bufs
    for chunk in range(RPT // CR):
        r0 = wid * RPT + chunk * CR
        pltpu.sync_copy(partials.at[0, pl.ds(r0, CR)], pa)
        pltpu.sync_copy(partials.at[1, pl.ds(r0, CR)], pb)
        pltpu.sync_copy(emb.at[pl.ds(r0, CR)], pe)

        def vec(k, carry):
            sl = pl.ds(k * L, L)
            deg = pa[sl] + pb[sl]
            dv1[sl] = jnp.where(deg > 0.0, _rsqrt(deg), 0.0)
            return carry

        lax.fori_loop(0, CR // L, vec, 0)

        def row(i, carry):
            dv = plsc.load_gather(dv1, [jnp.full((L,), i, jnp.int32)])
            odinv[i, :] = dv
            oy[i, :] = dv * pe[i, :]
            return carry

        lax.fori_loop(0, CR, row, 0)
        pltpu.sync_copy(odinv, dinv.at[pl.ds(r0, CR)])
        pltpu.sync_copy(oy, y0.at[pl.ds(r0, CR)])


@functools.partial(
    pl.kernel,
    out_type=(
        jax.ShapeDtypeStruct((NP, D), jnp.float32),
        jax.ShapeDtypeStruct((NP, D), jnp.float32),
    ),
    mesh=_mesh,
    compiler_params=_params,
    scratch_types=[pltpu.VMEM((CR, D), jnp.float32) for _ in range(6)],
)
def _comb2_kernel(partials, dinv, emb, s1, y1, *bufs):
    def body(i, p0, p1, dv, e):
        x1 = dv[i, :] * (p0[i, :] + p1[i, :])
        return e[i, :] + x1, dv[i, :] * x1

    _ew_loop(body, [partials.at[0], partials.at[1], dinv, emb], [s1, y1], bufs)


@functools.partial(
    pl.kernel,
    out_type=jax.ShapeDtypeStruct((NP, D), jnp.float32),
    mesh=_mesh,
    compiler_params=_params,
    scratch_types=[pltpu.VMEM((CR, D), jnp.float32) for _ in range(5)],
)
def _comb3_kernel(partials, dinv, s1, out, *bufs):
    third = jnp.float32(1.0 / 3.0)

    def body(i, p0, p1, dv, sv):
        return ((sv[i, :] + dv[i, :] * (p0[i, :] + p1[i, :])) * third,)

    _ew_loop(body, [partials.at[0], partials.at[1], dinv, s1], [out], bufs)


@functools.partial(
    pl.kernel,
    out_type=jax.ShapeDtypeStruct((PE,), jnp.float32),
    mesh=_mesh,
    compiler_params=_params,
    scratch_types=[
        pltpu.VMEM_SHARED((NP, D), jnp.float32),
        pltpu.VMEM((WD2,), jnp.int32),
        pltpu.VMEM((WD2,), jnp.int32),
        pltpu.VMEM((WD2,), jnp.int32),
        pltpu.VMEM((WD2,), jnp.int32),
        pltpu.VMEM((WD2, D), jnp.float32),
        pltpu.VMEM((WD2, D), jnp.float32),
        pltpu.VMEM((WD2, D), jnp.float32),
        pltpu.VMEM((WD2, D), jnp.float32),
        pltpu.VMEM((WD2,), jnp.float32),
        pltpu.SemaphoreType.DMA,
        pltpu.SemaphoreType.DMA,
    ],
)
def _dot_kernel(row1, col1, out, scores,
                stab, rowa, cola, rowb, colb, aa, ba, ab, bb, sb, sema, semb):
    _, _, wid = _ids()
    c, s, _w = _ids()
    iota = lax.iota(jnp.int32, L)

    r0 = s * RPC
    pltpu.sync_copy(out.at[pl.ds(r0, RPC)], stab.at[pl.ds(r0, RPC)])
    plsc.subcore_barrier()

    def fetch(w, rowx, colx, ax, bx, semx):
        base = wid * EPT + w * WD2
        pltpu.sync_copy(row1.at[pl.ds(base, WD2)], rowx)
        pltpu.sync_copy(col1.at[pl.ds(base, WD2)], colx)
        pltpu.async_copy(stab.at[rowx], ax, semx)
        pltpu.async_copy(stab.at[colx], bx, semx)

    def stage(w, rowx, colx, ax, bx, semx, rown, coln, an, bn, semn):
        @pl.when(w + 1 < NWD2)
        def _():
            fetch(w + 1, rown, coln, an, bn, semn)

        pltpu.make_async_copy(out.at[pl.ds(0, WD2)], ax, semx).wait()
        pltpu.make_async_copy(out.at[pl.ds(0, WD2)], bx, semx).wait()

        def group(g, carry2):
            rows = g * L + iota
            acc = jnp.zeros((L,), jnp.float32)
            for j in range(D):
                jv = jnp.full((L,), j, jnp.int32)
                va = plsc.load_gather(ax, [rows, jv])
                vb = plsc.load_gather(bx, [rows, jv])
                acc = acc + va * vb
            sb[pl.ds(g * L, L)] = acc
            return carry2

        lax.fori_loop(0, WD2 // L, group, 0)
        pltpu.sync_copy(sb, scores.at[pl.ds(wid * EPT + w * WD2, WD2)])

    fetch(0, rowa, cola, aa, ba, sema)

    def pair(w2, carry):
        w = 2 * w2
        stage(w, rowa, cola, aa, ba, sema, rowb, colb, ab, bb, semb)
        stage(w + 1, rowb, colb, ab, bb, semb, rowa, cola, aa, ba, sema)
        return carry

    lax.fori_loop(0, NWD2 // 2, pair, 0)


def kernel(edge_index, emb):
    pad = PE - E
    row = jnp.concatenate([edge_index[0], jnp.zeros((pad,), jnp.int32)])
    col = jnp.concatenate([edge_index[1], jnp.full((pad,), N, jnp.int32)])
    embp = jnp.pad(emb, ((0, NP - N), (0, 0)))

    deg_parts = _deg_kernel(col)
    dinv, y0 = _comb1_kernel(deg_parts, embp)
    p1 = _prop_kernel(row, col, y0)
    s1, y1 = _comb2_kernel(p1, dinv, embp)
    p2 = _prop_kernel(row, col, y1)
    out = _comb3_kernel(p2, dinv, s1)
    return _dot_kernel(row, col, out)[:E]


# speedup vs baseline: 1.1538x; 1.1538x over previous
import functools

import jax
import jax.numpy as jnp
from jax import lax
from jax.experimental import pallas as pl
from jax.experimental.pallas import tpu as pltpu
from jax.experimental.pallas import tpu_sc as plsc

N = 100000
D = 16
E = 3200000
NC = 2
NS = 16
NW = NC * NS
L = 16

NP = 100352
CH = 128
EPT = 100352
PE = NW * EPT
BPT = EPT // CH

KP = 4
WP = KP * CH
NWP = EPT // WP

KD = 8
WD = KD * CH
NWD = EPT // WD

KD2 = 2
WD2 = KD2 * CH
NWD2 = EPT // WD2

RPC = NP // NS
ZR = 224
RPT = NP // NW
CR = 784

_mesh = plsc.VectorSubcoreMesh(core_axis_name="c", subcore_axis_name="s")
_params = pltpu.CompilerParams(use_tc_tiling_on_sc=False, needs_layout_passes=False)


def _ids():
    c = lax.axis_index("c")
    s = lax.axis_index("s")
    return c, s, c * NS + s


def _fill(ref, rows, value):
    v = jnp.full((L,), value, jnp.float32)
    for r in range(rows):
        ref[r, :] = v


def _zero_acc(acc, zb, s):
    _fill(zb, ZR, 0.0)
    r0 = s * RPC
    for i in range(RPC // ZR):
        pltpu.sync_copy(zb.at[pl.ds(0, ZR)], acc.at[pl.ds(r0 + i * ZR, ZR)])


def _dump_acc(acc, partials, c, s):
    r0 = s * RPC
    pltpu.sync_copy(acc.at[pl.ds(r0, RPC)], partials.at[c, pl.ds(r0, RPC)])


@functools.partial(
    pl.kernel,
    out_type=jax.ShapeDtypeStruct((NC, NP), jnp.float32),
    mesh=_mesh,
    compiler_params=_params,
    scratch_types=[
        pltpu.VMEM_SHARED((NP,), jnp.float32),
        pltpu.VMEM((WD,), jnp.int32),
        pltpu.VMEM((WD,), jnp.int32),
        pltpu.VMEM((WD,), jnp.float32),
        pltpu.VMEM((RPC // 2,), jnp.float32),
        pltpu.SemaphoreType.DMA,
        pltpu.SemaphoreType.DMA,
    ],
)
def _deg_kernel(col1, partials, acc, cola, colb, ones, zb, ssema, ssemb):
    c, s, wid = _ids()
    for r in range(RPC // 2 // L):
        zb[pl.ds(r * L, L)] = jnp.zeros((L,), jnp.float32)
    r0 = s * RPC
    pltpu.sync_copy(zb, acc.at[pl.ds(r0, RPC // 2)])
    pltpu.sync_copy(zb, acc.at[pl.ds(r0 + RPC // 2, RPC // 2)])
    for r in range(WD // L):
        ones[pl.ds(r * L, L)] = jnp.full((L,), 1.0, jnp.float32)
    plsc.subcore_barrier()

    def load_idx(w, colx):
        base = wid * EPT + w * WD
        pltpu.sync_copy(col1.at[pl.ds(base, WD)], colx)

    def stage(w, colx, ssemx, coln, ssemn):
        @pl.when(w > 0)
        def _():
            pltpu.make_async_copy(ones, acc.at[coln], ssemn).wait()

        @pl.when(w + 1 < NWD)
        def _():
            load_idx(w + 1, coln)

        pltpu.async_copy(ones, acc.at[colx], ssemx, add=True)

    load_idx(0, cola)

    def pair(w2, carry):
        w = 2 * w2
        stage(w, cola, ssema, colb, ssemb)
        stage(w + 1, colb, ssemb, cola, ssema)
        return carry

    lax.fori_loop(0, NWD // 2, pair, 0)
    pltpu.make_async_copy(ones, acc.at[colb], ssemb).wait()
    plsc.subcore_barrier()
    pltpu.sync_copy(acc.at[pl.ds(r0, RPC)], partials.at[c, pl.ds(r0, RPC)])


@functools.partial(
    pl.kernel,
    out_type=jax.ShapeDtypeStruct((NC, NP, D), jnp.float32),
    mesh=_mesh,
    compiler_params=_params,
    scratch_types=[
        pltpu.VMEM_SHARED((NP, D), jnp.float32),
        pltpu.VMEM((WP,), jnp.int32),
        pltpu.VMEM((WP,), jnp.int32),
        pltpu.VMEM((WP,), jnp.int32),
        pltpu.VMEM((WP,), jnp.int32),
        pltpu.VMEM((WP, D), jnp.float32),
        pltpu.VMEM((WP, D), jnp.float32),
        pltpu.SemaphoreType.DMA,
        pltpu.SemaphoreType.DMA,
        pltpu.SemaphoreType.DMA,
        pltpu.SemaphoreType.DMA,
    ],
)
def _prop_kernel(row1, col1, ytab, partials,
                 acc, rowa, cola, rowb, colb, msga, msgb,
                 sema, semb, ssema, ssemb):
    c, s, wid = _ids()
    _zero_acc(acc, msga, s)
    plsc.subcore_barrier()

    def fetch(w, rowx, msgx, semx):
        base = wid * EPT + w * WP
        pltpu.sync_copy(row1.at[pl.ds(base, WP)], rowx)
        for j in range(KP):
            pltpu.async_copy(
                ytab.at[rowx.at[pl.ds(j * CH, CH)]],
                msgx.at[pl.ds(j * CH, CH)], semx,
            )

    def load_cols(w, colx):
        base = wid * EPT + w * WP
        pltpu.sync_copy(col1.at[pl.ds(base, WP)], colx)

    def stage(w, rowx, colx, msgx, semx, ssemx, rown, coln, msgn, semn, ssemn):
        @pl.when(w > 0)
        def _():
            pltpu.make_async_copy(msgn, acc.at[coln], ssemn).wait()

        @pl.when(w + 1 < NWP)
        def _():
            fetch(w + 1, rown, msgn, semn)

        pltpu.make_async_copy(ytab.at[pl.ds(0, WP)], msgx, semx).wait()
        load_cols(w, colx)
        pltpu.async_copy(msgx, acc.at[colx], ssemx, add=True)

    fetch(0, rowa, msga, sema)

    def pair(w2, carry):
        w = 2 * w2
        stage(w, rowa, cola, msga, sema, ssema, rowb, colb, msgb, semb, ssemb)
        stage(w + 1, rowb, colb, msgb, semb, ssemb, rowa, cola, msga, sema, ssema)
        return carry

    lax.fori_loop(0, NWP // 2, pair, 0)
    pltpu.make_async_copy(msgb, acc.at[colb], ssemb).wait()
    plsc.subcore_barrier()
    _dump_acc(acc, partials, c, s)


def _ew_loop(body, in_hbm, out_hbm, bufs):
    _, _, wid = _ids()
    n_in = len(in_hbm)
    in_bufs, out_bufs = bufs[:n_in], bufs[n_in:]
    for chunk in range(RPT // CR):
        r0 = wid * RPT + chunk * CR
        for ref, buf in zip(in_hbm, in_bufs):
            pltpu.sync_copy(ref.at[pl.ds(r0, CR)], buf)

        def row(i, carry):
            outs = body(i, *in_bufs)
            for o, b in zip(outs, out_bufs):
                b[i, :] = o
            return carry

        lax.fori_loop(0, CR, row, 0)
        for ref, buf in zip(out_hbm, out_bufs):
            pltpu.sync_copy(buf, ref.at[pl.ds(r0, CR)])


def _rsqrt(x):
    i = plsc.bitcast(x, jnp.int32)
    i = jnp.int32(0x5F3759DF) - lax.shift_right_logical(i, 1)
    y = plsc.bitcast(i, jnp.float32)
    half = x * 0.5
    for _ in range(4):
        y = y * (1.5 - half * y * y)
    return y


@functools.partial(
    pl.kernel,
    out_type=(
        jax.ShapeDtypeStruct((NP, D), jnp.float32),
        jax.ShapeDtypeStruct((NP, D), jnp.float32),
    ),
    mesh=_mesh,
    compiler_params=_params,
    scratch_types=[
        pltpu.VMEM((CR,), jnp.float32),
        pltpu.VMEM((CR,), jnp.float32),
        pltpu.VMEM((CR,), jnp.float32),
        pltpu.VMEM((CR, D), jnp.float32),
        pltpu.VMEM((CR, D), jnp.float32),
        pltpu.VMEM((CR, D), jnp.float32),
    ],
)
def _comb1_kernel(partials, emb, dinv, y0, *bufs):
    _, _, wid = _ids()
    pa, pb, dv1, pe, odinv, oy = bufs
    for chunk in range(RPT // CR):
        r0 = wid * RPT + chunk * CR
        pltpu.sync_copy(partials.at[0, pl.ds(r0, CR)], pa)
        pltpu.sync_copy(partials.at[1, pl.ds(r0, CR)], pb)
        pltpu.sync_copy(emb.at[pl.ds(r0, CR)], pe)

        def vec(k, carry):
            sl = pl.ds(k * L, L)
            deg = pa[sl] + pb[sl]
            dv1[sl] = jnp.where(deg > 0.0, _rsqrt(deg), 0.0)
            return carry

        lax.fori_loop(0, CR // L, vec, 0)

        def row(i, carry):
            dv = plsc.load_gather(dv1, [jnp.full((L,), i, jnp.int32)])
            odinv[i, :] = dv
            oy[i, :] = dv * pe[i, :]
            return carry

        lax.fori_loop(0, CR, row, 0)
        pltpu.sync_copy(odinv, dinv.at[pl.ds(r0, CR)])
        pltpu.sync_copy(oy, y0.at[pl.ds(r0, CR)])


@functools.partial(
    pl.kernel,
    out_type=(
        jax.ShapeDtypeStruct((NP, D), jnp.float32),
        jax.ShapeDtypeStruct((NP, D), jnp.float32),
    ),
    mesh=_mesh,
    compiler_params=_params,
    scratch_types=[pltpu.VMEM((CR, D), jnp.float32) for _ in range(6)],
)
def _comb2_kernel(partials, dinv, emb, s1, y1, *bufs):
    def body(i, p0, p1, dv, e):
        x1 = dv[i, :] * (p0[i, :] + p1[i, :])
        return e[i, :] + x1, dv[i, :] * x1

    _ew_loop(body, [partials.at[0], partials.at[1], dinv, emb], [s1, y1], bufs)


@functools.partial(
    pl.kernel,
    out_type=jax.ShapeDtypeStruct((NP, D), jnp.float32),
    mesh=_mesh,
    compiler_params=_params,
    scratch_types=[pltpu.VMEM((CR, D), jnp.float32) for _ in range(5)],
)
def _comb3_kernel(partials, dinv, s1, out, *bufs):
    third = jnp.float32(1.0 / 3.0)

    def body(i, p0, p1, dv, sv):
        return ((sv[i, :] + dv[i, :] * (p0[i, :] + p1[i, :])) * third,)

    _ew_loop(body, [partials.at[0], partials.at[1], dinv, s1], [out], bufs)


@functools.partial(
    pl.kernel,
    out_type=jax.ShapeDtypeStruct((PE,), jnp.float32),
    mesh=_mesh,
    compiler_params=_params,
    scratch_types=[
        pltpu.VMEM((WD,), jnp.int32),
        pltpu.VMEM((WD,), jnp.int32),
        pltpu.VMEM((WD,), jnp.int32),
        pltpu.VMEM((WD,), jnp.int32),
        pltpu.VMEM((WD, D), jnp.float32),
        pltpu.VMEM((WD, D), jnp.float32),
        pltpu.VMEM((WD, D), jnp.float32),
        pltpu.VMEM((WD, D), jnp.float32),
        pltpu.VMEM((WD,), jnp.float32),
        pltpu.SemaphoreType.DMA,
        pltpu.SemaphoreType.DMA,
    ],
)
def _dot_kernel(row1, col1, out, scores,
                rowa, cola, rowb, colb, aa, ba, ab, bb, sb, sema, semb):
    _, _, wid = _ids()
    iota = lax.iota(jnp.int32, L)

    def fetch(w, rowx, colx, ax, bx, semx):
        base = wid * EPT + w * WD
        pltpu.sync_copy(row1.at[pl.ds(base, WD)], rowx)
        pltpu.sync_copy(col1.at[pl.ds(base, WD)], colx)
        pltpu.async_copy(out.at[rowx], ax, semx)
        pltpu.async_copy(out.at[colx], bx, semx)

    def stage(w, rowx, colx, ax, bx, semx, rown, coln, an, bn, semn):
        @pl.when(w + 1 < NWD)
        def _():
            fetch(w + 1, rown, coln, an, bn, semn)

        pltpu.make_async_copy(out.at[pl.ds(0, WD)], ax, semx).wait()
        pltpu.make_async_copy(out.at[pl.ds(0, WD)], bx, semx).wait()

        def group(g, carry2):
            rows = g * L + iota
            acc = jnp.zeros((L,), jnp.float32)
            for j in range(D):
                jv = jnp.full((L,), j, jnp.int32)
                va = plsc.load_gather(ax, [rows, jv])
                vb = plsc.load_gather(bx, [rows, jv])
                acc = acc + va * vb
            sb[pl.ds(g * L, L)] = acc
            return carry2

        lax.fori_loop(0, WD // L, group, 0)
        pltpu.sync_copy(sb, scores.at[pl.ds(wid * EPT + w * WD, WD)])

    fetch(0, rowa, cola, aa, ba, sema)

    def pair(w2, carry):
        w = 2 * w2
        stage(w, rowa, cola, aa, ba, sema, rowb, colb, ab, bb, semb)
        stage(w + 1, rowb, colb, ab, bb, semb, rowa, cola, aa, ba, sema)
        return carry

    lax.fori_loop(0, NWD // 2, pair, 0)


def kernel(edge_index, emb):
    pad = PE - E
    row = jnp.concatenate([edge_index[0], jnp.zeros((pad,), jnp.int32)])
    col = jnp.concatenate([edge_index[1], jnp.full((pad,), N, jnp.int32)])
    embp = jnp.pad(emb, ((0, NP - N), (0, 0)))

    deg_parts = _deg_kernel(col)
    dinv, y0 = _comb1_kernel(deg_parts, embp)
    p1 = _prop_kernel(row, col, y0)
    s1, y1 = _comb2_kernel(p1, dinv, embp)
    p2 = _prop_kernel(row, col, y1)
    out = _comb3_kernel(p2, dinv, s1)
    return _dot_kernel(row, col, out)[:E]


# speedup vs baseline: 1.2626x; 1.0944x over previous
import functools

import jax
import jax.numpy as jnp
from jax import lax
from jax.experimental import pallas as pl
from jax.experimental.pallas import tpu as pltpu
from jax.experimental.pallas import tpu_sc as plsc

N = 100000
D = 16
E = 3200000
NC = 2
NS = 16
NW = NC * NS
L = 16

NP = 100352
CH = 128
EPT = 100352
PE = NW * EPT
BPT = EPT // CH

KP = 4
WP = KP * CH
NWP = EPT // WP

KD = 8
WD = KD * CH
NWD = EPT // WD

WDEG = 3584
NWDEG = EPT // WDEG
NBLK = PE // WP

RPC = NP // NS
ZR = 224
RPT = NP // NW
CR = 784

_mesh = plsc.VectorSubcoreMesh(core_axis_name="c", subcore_axis_name="s")
_params = pltpu.CompilerParams(use_tc_tiling_on_sc=False, needs_layout_passes=False)


def _ids():
    c = lax.axis_index("c")
    s = lax.axis_index("s")
    return c, s, c * NS + s


def _fill(ref, rows, value):
    v = jnp.full((L,), value, jnp.float32)
    for r in range(rows):
        ref[r, :] = v


def _zero_acc(acc, zb, s):
    _fill(zb, ZR, 0.0)
    r0 = s * RPC
    for i in range(RPC // ZR):
        pltpu.sync_copy(zb.at[pl.ds(0, ZR)], acc.at[pl.ds(r0 + i * ZR, ZR)])


def _dump_acc(acc, partials, c, s):
    r0 = s * RPC
    pltpu.sync_copy(acc.at[pl.ds(r0, RPC)], partials.at[c, pl.ds(r0, RPC)])


@functools.partial(
    pl.kernel,
    out_type=jax.ShapeDtypeStruct((NC, NP), jnp.float32),
    mesh=_mesh,
    compiler_params=_params,
    scratch_types=[
        pltpu.VMEM_SHARED((NP,), jnp.float32),
        pltpu.VMEM((WDEG,), jnp.int32),
        pltpu.VMEM((WDEG,), jnp.int32),
        pltpu.VMEM((WDEG,), jnp.float32),
        pltpu.VMEM((RPC // 2,), jnp.float32),
        pltpu.SemaphoreType.DMA,
        pltpu.SemaphoreType.DMA,
    ],
)
def _deg_kernel(col1, partials, acc, cola, colb, ones, zb, ssema, ssemb):
    c, s, wid = _ids()
    for r in range(RPC // 2 // L):
        zb[pl.ds(r * L, L)] = jnp.zeros((L,), jnp.float32)
    r0 = s * RPC
    pltpu.sync_copy(zb, acc.at[pl.ds(r0, RPC // 2)])
    pltpu.sync_copy(zb, acc.at[pl.ds(r0 + RPC // 2, RPC // 2)])
    for r in range(WDEG // L):
        ones[pl.ds(r * L, L)] = jnp.full((L,), 1.0, jnp.float32)
    plsc.subcore_barrier()

    def load_idx(w, colx):
        base = wid * EPT + w * WDEG
        pltpu.sync_copy(col1.at[pl.ds(base, WDEG)], colx)

    def stage(w, colx, ssemx, coln, ssemn):
        @pl.when(w > 0)
        def _():
            pltpu.make_async_copy(ones, acc.at[coln], ssemn).wait()

        @pl.when(w + 1 < NWDEG)
        def _():
            load_idx(w + 1, coln)

        pltpu.async_copy(ones, acc.at[colx], ssemx, add=True)

    load_idx(0, cola)

    def pair(w2, carry):
        w = 2 * w2
        stage(w, cola, ssema, colb, ssemb)
        stage(w + 1, colb, ssemb, cola, ssema)
        return carry

    lax.fori_loop(0, NWDEG // 2, pair, 0)
    pltpu.make_async_copy(ones, acc.at[colb], ssemb).wait()
    plsc.subcore_barrier()
    pltpu.sync_copy(acc.at[pl.ds(r0, RPC)], partials.at[c, pl.ds(r0, RPC)])


@functools.partial(
    pl.kernel,
    out_type=jax.ShapeDtypeStruct((NC, NP, D), jnp.float32),
    mesh=_mesh,
    compiler_params=_params,
    scratch_types=[
        pltpu.VMEM_SHARED((NP, D), jnp.float32),
        pltpu.VMEM((2, WP), jnp.int32),
        pltpu.VMEM((2, WP), jnp.int32),
        pltpu.VMEM((WP, D), jnp.float32),
        pltpu.VMEM((WP, D), jnp.float32),
        pltpu.SemaphoreType.DMA,
        pltpu.SemaphoreType.DMA,
        pltpu.SemaphoreType.DMA,
        pltpu.SemaphoreType.DMA,
    ],
)
def _prop_kernel(rc, ytab, partials,
                 acc, rca, rcb, msga, msgb,
                 sema, semb, ssema, ssemb):
    c, s, wid = _ids()
    _zero_acc(acc, msga, s)
    plsc.subcore_barrier()

    def fetch(w, rcx, msgx, semx):
        pltpu.sync_copy(rc.at[wid * NWP + w], rcx)
        pltpu.async_copy(ytab.at[rcx.at[0]], msgx, semx)

    def stage(w, rcx, msgx, semx, ssemx, rcn, msgn, semn, ssemn):
        @pl.when(w > 0)
        def _():
            pltpu.make_async_copy(msgn, acc.at[rcn.at[1]], ssemn).wait()

        @pl.when(w + 1 < NWP)
        def _():
            fetch(w + 1, rcn, msgn, semn)

        pltpu.make_async_copy(ytab.at[pl.ds(0, WP)], msgx, semx).wait()
        pltpu.async_copy(msgx, acc.at[rcx.at[1]], ssemx, add=True)

    fetch(0, rca, msga, sema)

    def pair(w2, carry):
        w = 2 * w2
        stage(w, rca, msga, sema, ssema, rcb, msgb, semb, ssemb)
        stage(w + 1, rcb, msgb, semb, ssemb, rca, msga, sema, ssema)
        return carry

    lax.fori_loop(0, NWP // 2, pair, 0)
    pltpu.make_async_copy(msgb, acc.at[rcb.at[1]], ssemb).wait()
    plsc.subcore_barrier()
    _dump_acc(acc, partials, c, s)


def _ew_loop(body, in_hbm, out_hbm, bufs):
    _, _, wid = _ids()
    n_in = len(in_hbm)
    in_bufs, out_bufs = bufs[:n_in], bufs[n_in:]
    for chunk in range(RPT // CR):
        r0 = wid * RPT + chunk * CR
        for ref, buf in zip(in_hbm, in_bufs):
            pltpu.sync_copy(ref.at[pl.ds(r0, CR)], buf)

        def row(i, carry):
            outs = body(i, *in_bufs)
            for o, b in zip(outs, out_bufs):
                b[i, :] = o
            return carry

        lax.fori_loop(0, CR, row, 0)
        for ref, buf in zip(out_hbm, out_bufs):
            pltpu.sync_copy(buf, ref.at[pl.ds(r0, CR)])


def _rsqrt(x):
    i = plsc.bitcast(x, jnp.int32)
    i = jnp.int32(0x5F3759DF) - lax.shift_right_logical(i, 1)
    y = plsc.bitcast(i, jnp.float32)
    half = x * 0.5
    for _ in range(4):
        y = y * (1.5 - half * y * y)
    return y


@functools.partial(
    pl.kernel,
    out_type=(
        jax.ShapeDtypeStruct((NP, D), jnp.float32),
        jax.ShapeDtypeStruct((NP, D), jnp.float32),
    ),
    mesh=_mesh,
    compiler_params=_params,
    scratch_types=[
        pltpu.VMEM((CR,), jnp.float32),
        pltpu.VMEM((CR,), jnp.float32),
        pltpu.VMEM((CR,), jnp.float32),
        pltpu.VMEM((CR, D), jnp.float32),
        pltpu.VMEM((CR, D), jnp.float32),
        pltpu.VMEM((CR, D), jnp.float32),
    ],
)
def _comb1_kernel(partials, emb, dinv, y0, *bufs):
    _, _, wid = _ids()
    pa, pb, dv1, pe, odinv, oy = bufs
    for chunk in range(RPT // CR):
        r0 = wid * RPT + chunk * CR
        pltpu.sync_copy(partials.at[0, pl.ds(r0, CR)], pa)
        pltpu.sync_copy(partials.at[1, pl.ds(r0, CR)], pb)
        pltpu.sync_copy(emb.at[pl.ds(r0, CR)], pe)

        def vec(k, carry):
            sl = pl.ds(k * L, L)
            deg = pa[sl] + pb[sl]
            dv1[sl] = jnp.where(deg > 0.0, _rsqrt(deg), 0.0)
            return carry

        lax.fori_loop(0, CR // L, vec, 0)

        def row(i, carry):
            dv = plsc.load_gather(dv1, [jnp.full((L,), i, jnp.int32)])
            odinv[i, :] = dv
            oy[i, :] = dv * pe[i, :]
            return carry

        lax.fori_loop(0, CR, row, 0)
        pltpu.sync_copy(odinv, dinv.at[pl.ds(r0, CR)])
        pltpu.sync_copy(oy, y0.at[pl.ds(r0, CR)])


@functools.partial(
    pl.kernel,
    out_type=(
        jax.ShapeDtypeStruct((NP, D), jnp.float32),
        jax.ShapeDtypeStruct((NP, D), jnp.float32),
    ),
    mesh=_mesh,
    compiler_params=_params,
    scratch_types=[pltpu.VMEM((CR, D), jnp.float32) for _ in range(6)],
)
def _comb2_kernel(partials, dinv, emb, s1, y1, *bufs):
    def body(i, p0, p1, dv, e):
        x1 = dv[i, :] * (p0[i, :] + p1[i, :])
        return e[i, :] + x1, dv[i, :] * x1

    _ew_loop(body, [partials.at[0], partials.at[1], dinv, emb], [s1, y1], bufs)


@functools.partial(
    pl.kernel,
    out_type=jax.ShapeDtypeStruct((NP, D), jnp.float32),
    mesh=_mesh,
    compiler_params=_params,
    scratch_types=[pltpu.VMEM((CR, D), jnp.float32) for _ in range(5)],
)
def _comb3_kernel(partials, dinv, s1, out, *bufs):
    third = jnp.float32(1.0 / 3.0)

    def body(i, p0, p1, dv, sv):
        return ((sv[i, :] + dv[i, :] * (p0[i, :] + p1[i, :])) * third,)

    _ew_loop(body, [partials.at[0], partials.at[1], dinv, s1], [out], bufs)


@functools.partial(
    pl.kernel,
    out_type=jax.ShapeDtypeStruct((PE,), jnp.float32),
    mesh=_mesh,
    compiler_params=_params,
    scratch_types=[
        pltpu.VMEM((2, 2, WP), jnp.int32),
        pltpu.VMEM((2, 2, WP), jnp.int32),
        pltpu.VMEM((WD, D), jnp.float32),
        pltpu.VMEM((WD, D), jnp.float32),
        pltpu.VMEM((WD, D), jnp.float32),
        pltpu.VMEM((WD, D), jnp.float32),
        pltpu.VMEM((WD,), jnp.float32),
        pltpu.SemaphoreType.DMA,
        pltpu.SemaphoreType.DMA,
    ],
)
def _dot_kernel(rc, out, scores,
                rca, rcb, aa, ba, ab, bb, sb, sema, semb):
    _, _, wid = _ids()
    iota = lax.iota(jnp.int32, L)

    def fetch(w, rcx, ax, bx, semx):
        pltpu.sync_copy(rc.at[pl.ds(wid * NWP + 2 * w, 2)], rcx)
        for h in range(2):
            pltpu.async_copy(out.at[rcx.at[h, 0]], ax.at[pl.ds(h * WP, WP)], semx)
            pltpu.async_copy(out.at[rcx.at[h, 1]], bx.at[pl.ds(h * WP, WP)], semx)

    def stage(w, rcx, ax, bx, semx, rcn, an, bn, semn):
        @pl.when(w + 1 < NWD)
        def _():
            fetch(w + 1, rcn, an, bn, semn)

        pltpu.make_async_copy(out.at[pl.ds(0, WD)], ax, semx).wait()
        pltpu.make_async_copy(out.at[pl.ds(0, WD)], bx, semx).wait()

        def group(g, carry2):
            rows = g * L + iota
            acc = jnp.zeros((L,), jnp.float32)
            for j in range(D):
                jv = jnp.full((L,), j, jnp.int32)
                va = plsc.load_gather(ax, [rows, jv])
                vb = plsc.load_gather(bx, [rows, jv])
                acc = acc + va * vb
            sb[pl.ds(g * L, L)] = acc
            return carry2

        lax.fori_loop(0, WD // L, group, 0)
        pltpu.sync_copy(sb, scores.at[pl.ds(wid * EPT + w * WD, WD)])

    fetch(0, rca, aa, ba, sema)

    def pair(w2, carry):
        w = 2 * w2
        stage(w, rca, aa, ba, sema, rcb, ab, bb, semb)
        stage(w + 1, rcb, ab, bb, semb, rca, aa, ba, sema)
        return carry

    lax.fori_loop(0, NWD // 2, pair, 0)


def kernel(edge_index, emb):
    pad = PE - E
    row = jnp.concatenate([edge_index[0], jnp.zeros((pad,), jnp.int32)])
    col = jnp.concatenate([edge_index[1], jnp.full((pad,), N, jnp.int32)])
    embp = jnp.pad(emb, ((0, NP - N), (0, 0)))
    rc = jnp.stack([row.reshape(NBLK, WP), col.reshape(NBLK, WP)], axis=1)

    deg_parts = _deg_kernel(col)
    dinv, y0 = _comb1_kernel(deg_parts, embp)
    p1 = _prop_kernel(rc, y0)
    s1, y1 = _comb2_kernel(p1, dinv, embp)
    p2 = _prop_kernel(rc, y1)
    out = _comb3_kernel(p2, dinv, s1)
    return _dot_kernel(rc, out)[:E]


# speedup vs baseline: 1.2699x; 1.0057x over previous
import functools

import jax
import jax.numpy as jnp
from jax import lax
from jax.experimental import pallas as pl
from jax.experimental.pallas import tpu as pltpu
from jax.experimental.pallas import tpu_sc as plsc

N = 100000
D = 16
E = 3200000
NC = 2
NS = 16
NW = NC * NS
L = 16

NP = 100352
CH = 128
EPT = 100352
PE = NW * EPT
BPT = EPT // CH

KP = 4
WP = KP * CH
NWP = EPT // WP

KD = 8
WD = KD * CH
NWD = EPT // WD

WDEG = 3584
NWDEG = EPT // WDEG
NBLK = PE // WP

RPC = NP // NS
ZR = 224
RPT = NP // NW
CR = 784

_mesh = plsc.VectorSubcoreMesh(core_axis_name="c", subcore_axis_name="s")
_params = pltpu.CompilerParams(use_tc_tiling_on_sc=False, needs_layout_passes=False)


def _ids():
    c = lax.axis_index("c")
    s = lax.axis_index("s")
    return c, s, c * NS + s


def _fill(ref, rows, value):
    v = jnp.full((L,), value, jnp.float32)
    for r in range(rows):
        ref[r, :] = v


def _zero_acc(acc, zb, s):
    _fill(zb, ZR, 0.0)
    r0 = s * RPC
    for i in range(RPC // ZR):
        pltpu.sync_copy(zb.at[pl.ds(0, ZR)], acc.at[pl.ds(r0 + i * ZR, ZR)])


def _dump_acc(acc, partials, c, s):
    r0 = s * RPC
    pltpu.sync_copy(acc.at[pl.ds(r0, RPC)], partials.at[c, pl.ds(r0, RPC)])


@functools.partial(
    pl.kernel,
    out_type=jax.ShapeDtypeStruct((NC, NP), jnp.float32),
    mesh=_mesh,
    compiler_params=_params,
    scratch_types=[
        pltpu.VMEM_SHARED((NP,), jnp.float32),
        pltpu.VMEM((WDEG,), jnp.int32),
        pltpu.VMEM((WDEG,), jnp.int32),
        pltpu.VMEM((WDEG,), jnp.float32),
        pltpu.VMEM((RPC // 2,), jnp.float32),
        pltpu.SemaphoreType.DMA,
        pltpu.SemaphoreType.DMA,
    ],
)
def _deg_kernel(col1, partials, acc, cola, colb, ones, zb, ssema, ssemb):
    c, s, wid = _ids()
    for r in range(RPC // 2 // L):
        zb[pl.ds(r * L, L)] = jnp.zeros((L,), jnp.float32)
    r0 = s * RPC
    pltpu.sync_copy(zb, acc.at[pl.ds(r0, RPC // 2)])
    pltpu.sync_copy(zb, acc.at[pl.ds(r0 + RPC // 2, RPC // 2)])
    for r in range(WDEG // L):
        ones[pl.ds(r * L, L)] = jnp.full((L,), 1.0, jnp.float32)
    plsc.subcore_barrier()

    def load_idx(w, colx):
        base = wid * EPT + w * WDEG
        pltpu.sync_copy(col1.at[pl.ds(base, WDEG)], colx)

    def stage(w, colx, ssemx, coln, ssemn):
        @pl.when(w > 0)
        def _():
            pltpu.make_async_copy(ones, acc.at[coln], ssemn).wait()

        @pl.when(w + 1 < NWDEG)
        def _():
            load_idx(w + 1, coln)

        pltpu.async_copy(ones, acc.at[colx], ssemx, add=True)

    load_idx(0, cola)

    def pair(w2, carry):
        w = 2 * w2
        stage(w, cola, ssema, colb, ssemb)
        stage(w + 1, colb, ssemb, cola, ssema)
        return carry

    lax.fori_loop(0, NWDEG // 2, pair, 0)
    pltpu.make_async_copy(ones, acc.at[colb], ssemb).wait()
    plsc.subcore_barrier()
    pltpu.sync_copy(acc.at[pl.ds(r0, RPC)], partials.at[c, pl.ds(r0, RPC)])


@functools.partial(
    pl.kernel,
    out_type=jax.ShapeDtypeStruct((NC, NP, D), jnp.float32),
    mesh=_mesh,
    compiler_params=_params,
    scratch_types=[
        pltpu.VMEM_SHARED((NP, D), jnp.float32),
        pltpu.VMEM((2, WP), jnp.int32),
        pltpu.VMEM((2, WP), jnp.int32),
        pltpu.VMEM((WP, D), jnp.float32),
        pltpu.VMEM((WP, D), jnp.float32),
        pltpu.SemaphoreType.DMA,
        pltpu.SemaphoreType.DMA,
        pltpu.SemaphoreType.DMA,
        pltpu.SemaphoreType.DMA,
    ],
)
def _prop_kernel(rc, ytab, partials,
                 acc, rca, rcb, msga, msgb,
                 sema, semb, ssema, ssemb):
    c, s, wid = _ids()
    _zero_acc(acc, msga, s)
    plsc.subcore_barrier()

    def fetch(w, rcx, msgx, semx):
        pltpu.sync_copy(rc.at[wid * NWP + w], rcx)
        pltpu.async_copy(ytab.at[rcx.at[0]], msgx, semx)

    def stage(w, rcx, msgx, semx, ssemx, rcn, msgn, semn, ssemn):
        @pl.when(w > 0)
        def _():
            pltpu.make_async_copy(msgn, acc.at[rcn.at[1]], ssemn).wait()

        @pl.when(w + 1 < NWP)
        def _():
            fetch(w + 1, rcn, msgn, semn)

        pltpu.make_async_copy(ytab.at[pl.ds(0, WP)], msgx, semx).wait()
        pltpu.async_copy(msgx, acc.at[rcx.at[1]], ssemx, add=True)

    fetch(0, rca, msga, sema)

    def pair(w2, carry):
        w = 2 * w2
        stage(w, rca, msga, sema, ssema, rcb, msgb, semb, ssemb)
        stage(w + 1, rcb, msgb, semb, ssemb, rca, msga, sema, ssema)
        return carry

    lax.fori_loop(0, NWP // 2, pair, 0)
    pltpu.make_async_copy(msgb, acc.at[rcb.at[1]], ssemb).wait()
    plsc.subcore_barrier()
    _dump_acc(acc, partials, c, s)


def _ew_loop(body, in_hbm, out_hbm, bufs):
    _, _, wid = _ids()
    n_in = len(in_hbm)
    in_bufs, out_bufs = bufs[:n_in], bufs[n_in:]
    for chunk in range(RPT // CR):
        r0 = wid * RPT + chunk * CR
        for ref, buf in zip(in_hbm, in_bufs):
            pltpu.sync_copy(ref.at[pl.ds(r0, CR)], buf)

        def row(i2, carry):
            for u in range(2):
                i = 2 * i2 + u
                outs = body(i, *in_bufs)
                for o, b in zip(outs, out_bufs):
                    b[i, :] = o
            return carry

        lax.fori_loop(0, CR // 2, row, 0)
        for ref, buf in zip(out_hbm, out_bufs):
            pltpu.sync_copy(buf, ref.at[pl.ds(r0, CR)])


def _rsqrt(x):
    i = plsc.bitcast(x, jnp.int32)
    i = jnp.int32(0x5F3759DF) - lax.shift_right_logical(i, 1)
    y = plsc.bitcast(i, jnp.float32)
    half = x * 0.5
    for _ in range(4):
        y = y * (1.5 - half * y * y)
    return y


@functools.partial(
    pl.kernel,
    out_type=(
        jax.ShapeDtypeStruct((NP, D), jnp.float32),
        jax.ShapeDtypeStruct((NP, D), jnp.float32),
    ),
    mesh=_mesh,
    compiler_params=_params,
    scratch_types=[
        pltpu.VMEM((CR,), jnp.float32),
        pltpu.VMEM((CR,), jnp.float32),
        pltpu.VMEM((CR,), jnp.float32),
        pltpu.VMEM((CR, D), jnp.float32),
        pltpu.VMEM((CR, D), jnp.float32),
        pltpu.VMEM((CR, D), jnp.float32),
    ],
)
def _comb1_kernel(partials, emb, dinv, y0, *bufs):
    _, _, wid = _ids()
    pa, pb, dv1, pe, odinv, oy = bufs
    for chunk in range(RPT // CR):
        r0 = wid * RPT + chunk * CR
        pltpu.sync_copy(partials.at[0, pl.ds(r0, CR)], pa)
        pltpu.sync_copy(partials.at[1, pl.ds(r0, CR)], pb)
        pltpu.sync_copy(emb.at[pl.ds(r0, CR)], pe)

        def vec(k, carry):
            sl = pl.ds(k * L, L)
            deg = pa[sl] + pb[sl]
            dv1[sl] = jnp.where(deg > 0.0, _rsqrt(deg), 0.0)
            return carry

        lax.fori_loop(0, CR // L, vec, 0)

        def row(i2, carry):
            for u in range(2):
                i = 2 * i2 + u
                dv = plsc.load_gather(dv1, [jnp.full((L,), i, jnp.int32)])
                odinv[i, :] = dv
                oy[i, :] = dv * pe[i, :]
            return carry

        lax.fori_loop(0, CR // 2, row, 0)
        pltpu.sync_copy(odinv, dinv.at[pl.ds(r0, CR)])
        pltpu.sync_copy(oy, y0.at[pl.ds(r0, CR)])


@functools.partial(
    pl.kernel,
    out_type=(
        jax.ShapeDtypeStruct((NP, D), jnp.float32),
        jax.ShapeDtypeStruct((NP, D), jnp.float32),
    ),
    mesh=_mesh,
    compiler_params=_params,
    scratch_types=[pltpu.VMEM((CR, D), jnp.float32) for _ in range(6)],
)
def _comb2_kernel(partials, dinv, emb, s1, y1, *bufs):
    def body(i, p0, p1, dv, e):
        x1 = dv[i, :] * (p0[i, :] + p1[i, :])
        return e[i, :] + x1, dv[i, :] * x1

    _ew_loop(body, [partials.at[0], partials.at[1], dinv, emb], [s1, y1], bufs)


@functools.partial(
    pl.kernel,
    out_type=jax.ShapeDtypeStruct((NP, D), jnp.float32),
    mesh=_mesh,
    compiler_params=_params,
    scratch_types=[pltpu.VMEM((CR, D), jnp.float32) for _ in range(5)],
)
def _comb3_kernel(partials, dinv, s1, out, *bufs):
    third = jnp.float32(1.0 / 3.0)

    def body(i, p0, p1, dv, sv):
        return ((sv[i, :] + dv[i, :] * (p0[i, :] + p1[i, :])) * third,)

    _ew_loop(body, [partials.at[0], partials.at[1], dinv, s1], [out], bufs)


@functools.partial(
    pl.kernel,
    out_type=jax.ShapeDtypeStruct((PE,), jnp.float32),
    mesh=_mesh,
    compiler_params=_params,
    scratch_types=[
        pltpu.VMEM((2, 2, WP), jnp.int32),
        pltpu.VMEM((2, 2, WP), jnp.int32),
        pltpu.VMEM((WD, D), jnp.float32),
        pltpu.VMEM((WD, D), jnp.float32),
        pltpu.VMEM((WD, D), jnp.float32),
        pltpu.VMEM((WD, D), jnp.float32),
        pltpu.VMEM((WD,), jnp.float32),
        pltpu.SemaphoreType.DMA,
        pltpu.SemaphoreType.DMA,
    ],
)
def _dot_kernel(rc, out, scores,
                rca, rcb, aa, ba, ab, bb, sb, sema, semb):
    _, _, wid = _ids()
    iota = lax.iota(jnp.int32, L)

    def fetch(w, rcx, ax, bx, semx):
        pltpu.sync_copy(rc.at[pl.ds(wid * NWP + 2 * w, 2)], rcx)
        for h in range(2):
            pltpu.async_copy(out.at[rcx.at[h, 0]], ax.at[pl.ds(h * WP, WP)], semx)
            pltpu.async_copy(out.at[rcx.at[h, 1]], bx.at[pl.ds(h * WP, WP)], semx)

    def stage(w, rcx, ax, bx, semx, rcn, an, bn, semn):
        @pl.when(w + 1 < NWD)
        def _():
            fetch(w + 1, rcn, an, bn, semn)

        pltpu.make_async_copy(out.at[pl.ds(0, WD)], ax, semx).wait()
        pltpu.make_async_copy(out.at[pl.ds(0, WD)], bx, semx).wait()

        def group(g, carry2):
            rows = g * L + iota
            acc = jnp.zeros((L,), jnp.float32)
            for j in range(D):
                jv = jnp.full((L,), j, jnp.int32)
                va = plsc.load_gather(ax, [rows, jv])
                vb = plsc.load_gather(bx, [rows, jv])
                acc = acc + va * vb
            sb[pl.ds(g * L, L)] = acc
            return carry2

        lax.fori_loop(0, WD // L, group, 0)
        pltpu.sync_copy(sb, scores.at[pl.ds(wid * EPT + w * WD, WD)])

    fetch(0, rca, aa, ba, sema)

    def pair(w2, carry):
        w = 2 * w2
        stage(w, rca, aa, ba, sema, rcb, ab, bb, semb)
        stage(w + 1, rcb, ab, bb, semb, rca, aa, ba, sema)
        return carry

    lax.fori_loop(0, NWD // 2, pair, 0)


def kernel(edge_index, emb):
    pad = PE - E
    row = jnp.concatenate([edge_index[0], jnp.zeros((pad,), jnp.int32)])
    col = jnp.concatenate([edge_index[1], jnp.full((pad,), N, jnp.int32)])
    embp = jnp.pad(emb, ((0, NP - N), (0, 0)))
    rc = jnp.stack([row.reshape(NBLK, WP), col.reshape(NBLK, WP)], axis=1)

    deg_parts = _deg_kernel(col)
    dinv, y0 = _comb1_kernel(deg_parts, embp)
    p1 = _prop_kernel(rc, y0)
    s1, y1 = _comb2_kernel(p1, dinv, embp)
    p2 = _prop_kernel(rc, y1)
    out = _comb3_kernel(p2, dinv, s1)
    return _dot_kernel(rc, out)[:E]
